# baseline (speedup 1.0000x reference)
import functools
import math

import jax
import jax.numpy as jnp
import numpy as np
from jax import lax
from jax.experimental import pallas as pl
from jax.experimental.pallas import tpu as pltpu
from jax.experimental.pallas import tpu_sc as plsc

INPUT_DIM = 3
NUM_LEVELS = 16
LEVEL_DIM = 2
BASE_RESOLUTION = 16
LOG2_HASHMAP_SIZE = 19

NC = 2
NS = 16
NW = NC * NS
LANES = 16


def _level_tables():
    offsets = []
    offset = 0
    max_params = 2 ** LOG2_HASHMAP_SIZE
    for i in range(NUM_LEVELS):
        resolution = int(np.ceil(BASE_RESOLUTION * 2.0 ** i))
        params_in_level = min(max_params, resolution ** INPUT_DIM)
        params_in_level = int(np.ceil(params_in_level / 8) * 8)
        offsets.append(offset)
        offset += params_in_level
    offsets.append(offset)

    params = []
    for lvl in range(NUM_LEVELS):
        size = offsets[lvl + 1] - offsets[lvl]
        scale = 2.0 ** lvl * BASE_RESOLUTION - 1.0
        resolution = int(math.ceil(scale)) + 1
        stride = 1
        coeffs = []
        use_stride = []
        for _ in range(INPUT_DIM):
            use_stride.append(stride <= size)
            coeffs.append(stride % (2 ** 32))
            stride = (stride * (resolution + 1)) % (2 ** 32)
        hashed = stride > size
        if hashed:
            c1 = int(np.int32(np.uint32(2654435761)))
            c2 = int(np.int32(np.uint32(805459861)))
            mode_add = False
        else:
            assert all(use_stride)
            c1 = int(np.int32(np.uint32(coeffs[1])))
            c2 = int(np.int32(np.uint32(coeffs[2])))
            mode_add = True
        params.append(dict(scale=float(scale), mask=size - 1,
                           off=offsets[lvl], add=mode_add, c1=c1, c2=c2))
    return params


_LEVELS = _level_tables()


def _make_grid_kernel(batch):
    chunk = batch // NW
    n = 256
    assert chunk % n == 0
    nsub = chunk // n
    groups = n // LANES
    m = 8 * n // 128

    f32 = jnp.float32
    i32 = jnp.int32

    def body(xh, yh, zh, eh, oh,
             xb, yb, zb, fb, idx0, idx1, lo0, lo1, rows0, rows1, accb,
             gsem0, gsem1, osem0):
        cid = lax.axis_index("c")
        sid = lax.axis_index("s")
        wid = sid * NC + cid
        base_w = wid * chunk
        iota = lax.iota(i32, LANES)
        idxb = (idx0, idx1)
        lob = (lo0, lo1)
        rowsb = (rows0, rows1)
        gsem = (gsem0, gsem1)

        pltpu.sync_copy(xh.at[pl.ds(base_w, chunk)], xb)
        pltpu.sync_copy(yh.at[pl.ds(base_w, chunk)], yb)
        pltpu.sync_copy(zh.at[pl.ds(base_w, chunk)], zb)

        def tbody(g, c):
            o = g * LANES
            for ref in (xb, yb, zb):
                v = ref[pl.ds(o, LANES)]
                ref[pl.ds(o, LANES)] = (v + f32(1.0)) * f32(0.5)
            return c
        lax.fori_loop(0, chunk // LANES, tbody, 0)

        def subchunk(s, carry):
            base = base_w + s * n
            sbase = s * n

            def compute_idx(lvl, p):
                prm = _LEVELS[lvl]
                scale = f32(prm["scale"])
                half = f32(0.5)
                c1 = i32(prm["c1"])
                c2 = i32(prm["c2"])
                mask = i32(prm["mask"])
                off = i32(prm["off"])
                iref = idxb[p]
                lref = lob[p]

                def cbody(g, c):
                    o = g * LANES
                    px = xb[pl.ds(sbase + o, LANES)] * scale + half
                    py = yb[pl.ds(sbase + o, LANES)] * scale + half
                    pz = zb[pl.ds(sbase + o, LANES)] * scale + half
                    pix = px.astype(i32)
                    piy = py.astype(i32)
                    piz = pz.astype(i32)
                    fb[p, 0, pl.ds(o, LANES)] = px - pix.astype(f32)
                    fb[p, 1, pl.ds(o, LANES)] = py - piy.astype(f32)
                    fb[p, 2, pl.ds(o, LANES)] = pz - piz.astype(f32)
                    ax = (pix, pix + i32(1))
                    by = (piy * c1, piy * c1 + c1)
                    cz = (piz * c2, piz * c2 + c2)
                    for c8 in range(8):
                        a = ax[c8 & 1]
                        b = by[(c8 >> 1) & 1]
                        cc = cz[(c8 >> 2) & 1]
                        if prm["add"]:
                            h = a + b + cc
                        else:
                            h = a ^ b ^ cc
                        glob = (h & mask) + off
                        p0 = ((glob >> 7) << 5) + ((glob & i32(127)) >> 3)
                        iref[pl.ds(c8 * n + o, LANES)] = p0
                        iref[pl.ds(8 * n + c8 * n + o, LANES)] = p0 + i32(16)
                        lref[pl.ds(c8 * n + o, LANES)] = glob & i32(7)
                    return c
                lax.fori_loop(0, groups, cbody, 0)

            def accumulate(lvl, p):
                rref = rowsb[p]
                lref = lob[p]

                def abody(g, c):
                    o = g * LANES
                    fx = fb[p, 0, pl.ds(o, LANES)]
                    fy = fb[p, 1, pl.ds(o, LANES)]
                    fz = fb[p, 2, pl.ds(o, LANES)]
                    gx = f32(1.0) - fx
                    gy = f32(1.0) - fy
                    gz = f32(1.0) - fz
                    wxy = (gx * gy, fx * gy, gx * fy, fx * fy)
                    jv = o + iota
                    a0 = None
                    a1 = None
                    for c8 in range(8):
                        w = wxy[c8 & 3] * (gz if c8 < 4 else fz)
                        lo = lref[pl.ds(c8 * n + o, LANES)]
                        v0 = plsc.load_gather(rref, [jv + c8 * n, lo])
                        v1 = plsc.load_gather(rref, [jv + (8 + c8) * n, lo])
                        t0 = w * v0
                        t1 = w * v1
                        a0 = t0 if a0 is None else a0 + t0
                        a1 = t1 if a1 is None else a1 + t1
                    accb[2 * lvl, pl.ds(o, LANES)] = a0
                    accb[2 * lvl + 1, pl.ds(o, LANES)] = a1
                    return c
                lax.fori_loop(0, groups, abody, 0)

            ghandles = [None, None]
            for lvl in range(NUM_LEVELS):
                p = lvl & 1
                compute_idx(lvl, p)
                ghandles[p] = pltpu.async_copy(eh.at[idxb[p]], rowsb[p],
                                               gsem[p])
                if lvl > 0:
                    q = (lvl - 1) & 1
                    ghandles[q].wait()
                    accumulate(lvl - 1, q)
            ghandles[1].wait()
            accumulate(NUM_LEVELS - 1, 1)
            pltpu.async_copy(
                accb, oh.at[pl.ds(0, 2 * NUM_LEVELS), pl.ds(base, n)],
                osem0).wait()
            return carry

        lax.fori_loop(0, nsub, subchunk, 0)

    mesh = plsc.VectorSubcoreMesh(core_axis_name="c", subcore_axis_name="s")
    return pl.kernel(
        body,
        out_type=jax.ShapeDtypeStruct((NUM_LEVELS * LEVEL_DIM, batch), f32),
        mesh=mesh,
        compiler_params=pltpu.CompilerParams(
            needs_layout_passes=False,
            use_tc_tiling_on_sc=False,
        ),
        scratch_types=[
            pltpu.VMEM((chunk,), f32),
            pltpu.VMEM((chunk,), f32),
            pltpu.VMEM((chunk,), f32),
            pltpu.VMEM((2, 3, n), f32),
            pltpu.VMEM((16 * n,), i32),
            pltpu.VMEM((16 * n,), i32),
            pltpu.VMEM((8 * n,), i32),
            pltpu.VMEM((8 * n,), i32),
            pltpu.VMEM((16 * n, 8), f32),
            pltpu.VMEM((16 * n, 8), f32),
            pltpu.VMEM((2 * NUM_LEVELS, n), f32),
            pltpu.SemaphoreType.DMA,
            pltpu.SemaphoreType.DMA,
            pltpu.SemaphoreType.DMA,
        ],
    )


@jax.jit
def kernel(inputs, embeddings):
    batch = inputs.shape[0]
    xt = inputs.T
    grid = _make_grid_kernel(batch)
    nrows = embeddings.shape[0]
    emb3 = jnp.transpose(embeddings.reshape(nrows // 128, 128, 2), (0, 2, 1))
    packed = emb3.reshape(nrows * 2 // 8, 8)
    out = grid(xt[0], xt[1], xt[2], packed)
    return out.T


# speedup vs baseline: 1.0170x; 1.0170x over previous
import functools
import math

import jax
import jax.numpy as jnp
import numpy as np
from jax import lax
from jax.experimental import pallas as pl
from jax.experimental.pallas import tpu as pltpu
from jax.experimental.pallas import tpu_sc as plsc

INPUT_DIM = 3
NUM_LEVELS = 16
LEVEL_DIM = 2
BASE_RESOLUTION = 16
LOG2_HASHMAP_SIZE = 19

NC = 2
NS = 16
NW = NC * NS
LANES = 16


def _level_tables():
    offsets = []
    offset = 0
    max_params = 2 ** LOG2_HASHMAP_SIZE
    for i in range(NUM_LEVELS):
        resolution = int(np.ceil(BASE_RESOLUTION * 2.0 ** i))
        params_in_level = min(max_params, resolution ** INPUT_DIM)
        params_in_level = int(np.ceil(params_in_level / 8) * 8)
        offsets.append(offset)
        offset += params_in_level
    offsets.append(offset)

    params = []
    for lvl in range(NUM_LEVELS):
        size = offsets[lvl + 1] - offsets[lvl]
        scale = 2.0 ** lvl * BASE_RESOLUTION - 1.0
        resolution = int(math.ceil(scale)) + 1
        stride = 1
        coeffs = []
        use_stride = []
        for _ in range(INPUT_DIM):
            use_stride.append(stride <= size)
            coeffs.append(stride % (2 ** 32))
            stride = (stride * (resolution + 1)) % (2 ** 32)
        hashed = stride > size
        if hashed:
            c1 = int(np.int32(np.uint32(2654435761)))
            c2 = int(np.int32(np.uint32(805459861)))
            mode_add = False
        else:
            assert all(use_stride)
            c1 = int(np.int32(np.uint32(coeffs[1])))
            c2 = int(np.int32(np.uint32(coeffs[2])))
            mode_add = True
        params.append(dict(scale=float(scale), mask=size - 1,
                           off=offsets[lvl], add=mode_add, c1=c1, c2=c2))
    return params


_LEVELS = _level_tables()


def _make_grid_kernel(batch):
    chunk = batch // NW
    n = 256
    assert chunk % n == 0
    nsub = chunk // n
    groups = n // LANES
    m = 8 * n // 128

    f32 = jnp.float32
    i32 = jnp.int32

    def body(xh, yh, zh, eh, oh,
             xb, yb, zb, fb, idx0, idx1, lo0, lo1, rows0, rows1, accb,
             gsem0, gsem1, gsem2, gsem3, osem0):
        cid = lax.axis_index("c")
        sid = lax.axis_index("s")
        wid = sid * NC + cid
        base_w = wid * chunk
        iota = lax.iota(i32, LANES)
        idxb = (idx0, idx1)
        lob = (lo0, lo1)
        rowsb = (rows0, rows1)
        gsem = (gsem0, gsem1)
        gsemb = (gsem2, gsem3)

        pltpu.sync_copy(xh.at[pl.ds(base_w, chunk)], xb)
        pltpu.sync_copy(yh.at[pl.ds(base_w, chunk)], yb)
        pltpu.sync_copy(zh.at[pl.ds(base_w, chunk)], zb)

        def tbody(g, c):
            o = g * LANES
            for ref in (xb, yb, zb):
                v = ref[pl.ds(o, LANES)]
                ref[pl.ds(o, LANES)] = (v + f32(1.0)) * f32(0.5)
            return c
        lax.fori_loop(0, chunk // LANES, tbody, 0)

        def subchunk(s, carry):
            base = base_w + s * n
            sbase = s * n

            def compute_idx(lvl, p):
                prm = _LEVELS[lvl]
                scale = f32(prm["scale"])
                half = f32(0.5)
                c1 = i32(prm["c1"])
                c2 = i32(prm["c2"])
                mask = i32(prm["mask"])
                off = i32(prm["off"])
                iref = idxb[p]
                lref = lob[p]

                def cbody(g, c):
                    o = g * LANES
                    px = xb[pl.ds(sbase + o, LANES)] * scale + half
                    py = yb[pl.ds(sbase + o, LANES)] * scale + half
                    pz = zb[pl.ds(sbase + o, LANES)] * scale + half
                    pix = px.astype(i32)
                    piy = py.astype(i32)
                    piz = pz.astype(i32)
                    fb[p, 0, pl.ds(o, LANES)] = px - pix.astype(f32)
                    fb[p, 1, pl.ds(o, LANES)] = py - piy.astype(f32)
                    fb[p, 2, pl.ds(o, LANES)] = pz - piz.astype(f32)
                    ax = (pix, pix + i32(1))
                    by = (piy * c1, piy * c1 + c1)
                    cz = (piz * c2, piz * c2 + c2)
                    for c8 in range(8):
                        a = ax[c8 & 1]
                        b = by[(c8 >> 1) & 1]
                        cc = cz[(c8 >> 2) & 1]
                        if prm["add"]:
                            h = a + b + cc
                        else:
                            h = a ^ b ^ cc
                        glob = (h & mask) + off
                        p0 = ((glob >> 7) << 5) + ((glob & i32(127)) >> 3)
                        iref[pl.ds(c8 * n + o, LANES)] = p0
                        iref[pl.ds(8 * n + c8 * n + o, LANES)] = p0 + i32(16)
                        lref[pl.ds(c8 * n + o, LANES)] = glob & i32(7)
                    return c
                lax.fori_loop(0, groups, cbody, 0)

            def accumulate(lvl, p):
                rref = rowsb[p]
                lref = lob[p]

                def abody(g, c):
                    o = g * LANES
                    fx = fb[p, 0, pl.ds(o, LANES)]
                    fy = fb[p, 1, pl.ds(o, LANES)]
                    fz = fb[p, 2, pl.ds(o, LANES)]
                    gx = f32(1.0) - fx
                    gy = f32(1.0) - fy
                    gz = f32(1.0) - fz
                    wxy = (gx * gy, fx * gy, gx * fy, fx * fy)
                    jv = o + iota
                    a0 = None
                    a1 = None
                    for c8 in range(8):
                        w = wxy[c8 & 3] * (gz if c8 < 4 else fz)
                        lo = lref[pl.ds(c8 * n + o, LANES)]
                        v0 = plsc.load_gather(rref, [jv + c8 * n, lo])
                        v1 = plsc.load_gather(rref, [jv + (8 + c8) * n, lo])
                        t0 = w * v0
                        t1 = w * v1
                        a0 = t0 if a0 is None else a0 + t0
                        a1 = t1 if a1 is None else a1 + t1
                    accb[2 * lvl, pl.ds(o, LANES)] = a0
                    accb[2 * lvl + 1, pl.ds(o, LANES)] = a1
                    return c
                lax.fori_loop(0, groups, abody, 0)

            ghandles = [None, None]
            for lvl in range(NUM_LEVELS):
                p = lvl & 1
                compute_idx(lvl, p)
                ha = pltpu.async_copy(
                    eh.at[idxb[p].at[pl.ds(0, 8 * n)]],
                    rowsb[p].at[pl.ds(0, 8 * n)], gsem[p])
                hb = pltpu.async_copy(
                    eh.at[idxb[p].at[pl.ds(8 * n, 8 * n)]],
                    rowsb[p].at[pl.ds(8 * n, 8 * n)], gsemb[p])
                ghandles[p] = (ha, hb)
                if lvl > 0:
                    q = (lvl - 1) & 1
                    ghandles[q][0].wait()
                    ghandles[q][1].wait()
                    accumulate(lvl - 1, q)
            ghandles[1][0].wait()
            ghandles[1][1].wait()
            accumulate(NUM_LEVELS - 1, 1)
            pltpu.async_copy(
                accb, oh.at[pl.ds(0, 2 * NUM_LEVELS), pl.ds(base, n)],
                osem0).wait()
            return carry

        lax.fori_loop(0, nsub, subchunk, 0)

    mesh = plsc.VectorSubcoreMesh(core_axis_name="c", subcore_axis_name="s")
    return pl.kernel(
        body,
        out_type=jax.ShapeDtypeStruct((NUM_LEVELS * LEVEL_DIM, batch), f32),
        mesh=mesh,
        compiler_params=pltpu.CompilerParams(
            needs_layout_passes=False,
            use_tc_tiling_on_sc=False,
        ),
        scratch_types=[
            pltpu.VMEM((chunk,), f32),
            pltpu.VMEM((chunk,), f32),
            pltpu.VMEM((chunk,), f32),
            pltpu.VMEM((2, 3, n), f32),
            pltpu.VMEM((16 * n,), i32),
            pltpu.VMEM((16 * n,), i32),
            pltpu.VMEM((8 * n,), i32),
            pltpu.VMEM((8 * n,), i32),
            pltpu.VMEM((16 * n, 8), f32),
            pltpu.VMEM((16 * n, 8), f32),
            pltpu.VMEM((2 * NUM_LEVELS, n), f32),
            pltpu.SemaphoreType.DMA,
            pltpu.SemaphoreType.DMA,
            pltpu.SemaphoreType.DMA,
            pltpu.SemaphoreType.DMA,
            pltpu.SemaphoreType.DMA,
        ],
    )


@jax.jit
def kernel(inputs, embeddings):
    batch = inputs.shape[0]
    xt = inputs.T
    grid = _make_grid_kernel(batch)
    nrows = embeddings.shape[0]
    emb3 = jnp.transpose(embeddings.reshape(nrows // 128, 128, 2), (0, 2, 1))
    packed = emb3.reshape(nrows * 2 // 8, 8)
    out = grid(xt[0], xt[1], xt[2], packed)
    return out.T


# speedup vs baseline: 1.1711x; 1.1516x over previous
import functools
import math

import jax
import jax.numpy as jnp
import numpy as np
from jax import lax
from jax.experimental import pallas as pl
from jax.experimental.pallas import tpu as pltpu
from jax.experimental.pallas import tpu_sc as plsc

INPUT_DIM = 3
NUM_LEVELS = 16
LEVEL_DIM = 2
BASE_RESOLUTION = 16
LOG2_HASHMAP_SIZE = 19

NC = 2
NS = 16
NW = NC * NS
LANES = 16


def _level_tables():
    offsets = []
    offset = 0
    max_params = 2 ** LOG2_HASHMAP_SIZE
    for i in range(NUM_LEVELS):
        resolution = int(np.ceil(BASE_RESOLUTION * 2.0 ** i))
        params_in_level = min(max_params, resolution ** INPUT_DIM)
        params_in_level = int(np.ceil(params_in_level / 8) * 8)
        offsets.append(offset)
        offset += params_in_level
    offsets.append(offset)

    params = []
    for lvl in range(NUM_LEVELS):
        size = offsets[lvl + 1] - offsets[lvl]
        scale = 2.0 ** lvl * BASE_RESOLUTION - 1.0
        resolution = int(math.ceil(scale)) + 1
        stride = 1
        coeffs = []
        use_stride = []
        for _ in range(INPUT_DIM):
            use_stride.append(stride <= size)
            coeffs.append(stride % (2 ** 32))
            stride = (stride * (resolution + 1)) % (2 ** 32)
        hashed = stride > size
        if hashed:
            c1 = int(np.int32(np.uint32(2654435761)))
            c2 = int(np.int32(np.uint32(805459861)))
            mode_add = False
        else:
            assert all(use_stride)
            c1 = int(np.int32(np.uint32(coeffs[1])))
            c2 = int(np.int32(np.uint32(coeffs[2])))
            mode_add = True
        params.append(dict(scale=float(scale), mask=size - 1,
                           off=offsets[lvl], add=mode_add, c1=c1, c2=c2))
    return params


_LEVELS = _level_tables()


def _make_grid_kernel(batch):
    chunk = batch // NW
    n = 256
    assert chunk % n == 0
    nsub = chunk // n
    groups = n // LANES
    m = 8 * n // 128

    f32 = jnp.float32
    i32 = jnp.int32

    def body(xh, yh, zh, eh, oh,
             xb, yb, zb, fb, idx0, idx1, lo0, lo1, rows0, rows1, accb, l0b,
             gsem0, gsem1, gsem2, gsem3, osem0):
        cid = lax.axis_index("c")
        sid = lax.axis_index("s")
        wid = sid * NC + cid
        base_w = wid * chunk
        iota = lax.iota(i32, LANES)
        idxb = (idx0, idx1)
        lob = (lo0, lo1)
        rowsb = (rows0, rows1)
        gsem = (gsem0, gsem1)
        gsemb = (gsem2, gsem3)

        pltpu.sync_copy(xh.at[pl.ds(base_w, chunk)], xb)
        pltpu.sync_copy(yh.at[pl.ds(base_w, chunk)], yb)
        pltpu.sync_copy(zh.at[pl.ds(base_w, chunk)], zb)
        pltpu.sync_copy(eh.at[pl.ds(0, 1024)], l0b)
        pltpu.async_copy(accb, oh.at[pl.ds(0, 2 * NUM_LEVELS),
                                     pl.ds(base_w, n)], osem0)

        def tbody(g, c):
            o = g * LANES
            for ref in (xb, yb, zb):
                v = ref[pl.ds(o, LANES)]
                ref[pl.ds(o, LANES)] = (v + f32(1.0)) * f32(0.5)
            return c
        lax.fori_loop(0, chunk // LANES, tbody, 0)

        def subchunk(s, carry):
            base = base_w + s * n
            sbase = s * n

            def compute_idx(lvl, p):
                prm = _LEVELS[lvl]
                scale = f32(prm["scale"])
                half = f32(0.5)
                c1 = i32(prm["c1"])
                c2 = i32(prm["c2"])
                mask = i32(prm["mask"])
                off = i32(prm["off"])
                iref = idxb[p]
                lref = lob[p]

                def cbody(g, c):
                    o = g * LANES
                    px = xb[pl.ds(sbase + o, LANES)] * scale + half
                    py = yb[pl.ds(sbase + o, LANES)] * scale + half
                    pz = zb[pl.ds(sbase + o, LANES)] * scale + half
                    pix = px.astype(i32)
                    piy = py.astype(i32)
                    piz = pz.astype(i32)
                    fb[p, 0, pl.ds(o, LANES)] = px - pix.astype(f32)
                    fb[p, 1, pl.ds(o, LANES)] = py - piy.astype(f32)
                    fb[p, 2, pl.ds(o, LANES)] = pz - piz.astype(f32)
                    ax = (pix, pix + i32(1))
                    by = (piy * c1, piy * c1 + c1)
                    cz = (piz * c2, piz * c2 + c2)
                    for c8 in range(8):
                        a = ax[c8 & 1]
                        b = by[(c8 >> 1) & 1]
                        cc = cz[(c8 >> 2) & 1]
                        if prm["add"]:
                            h = a + b + cc
                        else:
                            h = a ^ b ^ cc
                        glob = (h & mask) + off
                        p0 = ((glob >> 7) << 5) + ((glob & i32(127)) >> 3)
                        iref[pl.ds(c8 * n + o, LANES)] = p0
                        iref[pl.ds(8 * n + c8 * n + o, LANES)] = p0 + i32(16)
                        lref[pl.ds(c8 * n + o, LANES)] = glob & i32(7)
                    return c
                lax.fori_loop(0, groups, cbody, 0)

            def accumulate(lvl, p):
                rref = rowsb[p]
                lref = lob[p]
                iref = idxb[p]

                def abody(g, c):
                    o = g * LANES
                    fx = fb[p, 0, pl.ds(o, LANES)]
                    fy = fb[p, 1, pl.ds(o, LANES)]
                    fz = fb[p, 2, pl.ds(o, LANES)]
                    gx = f32(1.0) - fx
                    gy = f32(1.0) - fy
                    gz = f32(1.0) - fz
                    wxy = (gx * gy, fx * gy, gx * fy, fx * fy)
                    jv = o + iota
                    a0 = None
                    a1 = None
                    for c8 in range(8):
                        w = wxy[c8 & 3] * (gz if c8 < 4 else fz)
                        lo = lref[pl.ds(c8 * n + o, LANES)]
                        if lvl == 0:
                            pv = iref[pl.ds(c8 * n + o, LANES)]
                            v0 = plsc.load_gather(l0b, [pv, lo])
                            v1 = plsc.load_gather(l0b, [pv + i32(16), lo])
                        else:
                            v0 = plsc.load_gather(rref, [jv + c8 * n, lo])
                            v1 = plsc.load_gather(rref,
                                                  [jv + (8 + c8) * n, lo])
                        t0 = w * v0
                        t1 = w * v1
                        a0 = t0 if a0 is None else a0 + t0
                        a1 = t1 if a1 is None else a1 + t1
                    accb[2 * lvl, pl.ds(o, LANES)] = a0
                    accb[2 * lvl + 1, pl.ds(o, LANES)] = a1
                    return c
                lax.fori_loop(0, groups, abody, 0)

            ghandles = [None, None]
            for lvl in range(NUM_LEVELS):
                p = lvl & 1
                compute_idx(lvl, p)
                if lvl > 0:
                    ha = pltpu.async_copy(
                        eh.at[idxb[p].at[pl.ds(0, 8 * n)]],
                        rowsb[p].at[pl.ds(0, 8 * n)], gsem[p])
                    hb = pltpu.async_copy(
                        eh.at[idxb[p].at[pl.ds(8 * n, 8 * n)]],
                        rowsb[p].at[pl.ds(8 * n, 8 * n)], gsemb[p])
                    ghandles[p] = (ha, hb)
                if lvl == 1:
                    pltpu.make_async_copy(
                        accb, oh.at[pl.ds(0, 2 * NUM_LEVELS), pl.ds(base, n)],
                        osem0).wait()
                    accumulate(0, 0)
                elif lvl > 1:
                    q = (lvl - 1) & 1
                    ghandles[q][0].wait()
                    ghandles[q][1].wait()
                    accumulate(lvl - 1, q)
            ghandles[1][0].wait()
            ghandles[1][1].wait()
            accumulate(NUM_LEVELS - 1, 1)
            pltpu.async_copy(
                accb, oh.at[pl.ds(0, 2 * NUM_LEVELS), pl.ds(base, n)],
                osem0)
            return carry

        lax.fori_loop(0, nsub, subchunk, 0)
        pltpu.make_async_copy(
            accb,
            oh.at[pl.ds(0, 2 * NUM_LEVELS),
                  pl.ds(base_w + (nsub - 1) * n, n)],
            osem0).wait()

    mesh = plsc.VectorSubcoreMesh(core_axis_name="c", subcore_axis_name="s")
    return pl.kernel(
        body,
        out_type=jax.ShapeDtypeStruct((NUM_LEVELS * LEVEL_DIM, batch), f32),
        mesh=mesh,
        compiler_params=pltpu.CompilerParams(
            needs_layout_passes=False,
            use_tc_tiling_on_sc=False,
        ),
        scratch_types=[
            pltpu.VMEM((chunk,), f32),
            pltpu.VMEM((chunk,), f32),
            pltpu.VMEM((chunk,), f32),
            pltpu.VMEM((2, 3, n), f32),
            pltpu.VMEM((16 * n,), i32),
            pltpu.VMEM((16 * n,), i32),
            pltpu.VMEM((8 * n,), i32),
            pltpu.VMEM((8 * n,), i32),
            pltpu.VMEM((16 * n, 8), f32),
            pltpu.VMEM((16 * n, 8), f32),
            pltpu.VMEM((2 * NUM_LEVELS, n), f32),
            pltpu.VMEM((1024, 8), f32),
            pltpu.SemaphoreType.DMA,
            pltpu.SemaphoreType.DMA,
            pltpu.SemaphoreType.DMA,
            pltpu.SemaphoreType.DMA,
            pltpu.SemaphoreType.DMA,
        ],
    )


@jax.jit
def kernel(inputs, embeddings):
    batch = inputs.shape[0]
    xt = inputs.T
    grid = _make_grid_kernel(batch)
    nrows = embeddings.shape[0]
    emb3 = jnp.transpose(embeddings.reshape(nrows // 128, 128, 2), (0, 2, 1))
    packed = emb3.reshape(nrows * 2 // 8, 8)
    out = grid(xt[0], xt[1], xt[2], packed)
    return out.T


# speedup vs baseline: 1.7761x; 1.5166x over previous
import functools
import math

import jax
import jax.numpy as jnp
import numpy as np
from jax import lax
from jax.experimental import pallas as pl
from jax.experimental.pallas import tpu as pltpu
from jax.experimental.pallas import tpu_sc as plsc

INPUT_DIM = 3
NUM_LEVELS = 16
LEVEL_DIM = 2
BASE_RESOLUTION = 16
LOG2_HASHMAP_SIZE = 19

NC = 2
NS = 16
NW = NC * NS
LANES = 16


def _level_tables():
    offsets = []
    offset = 0
    max_params = 2 ** LOG2_HASHMAP_SIZE
    for i in range(NUM_LEVELS):
        resolution = int(np.ceil(BASE_RESOLUTION * 2.0 ** i))
        params_in_level = min(max_params, resolution ** INPUT_DIM)
        params_in_level = int(np.ceil(params_in_level / 8) * 8)
        offsets.append(offset)
        offset += params_in_level
    offsets.append(offset)

    params = []
    for lvl in range(NUM_LEVELS):
        size = offsets[lvl + 1] - offsets[lvl]
        scale = 2.0 ** lvl * BASE_RESOLUTION - 1.0
        resolution = int(math.ceil(scale)) + 1
        stride = 1
        coeffs = []
        use_stride = []
        for _ in range(INPUT_DIM):
            use_stride.append(stride <= size)
            coeffs.append(stride % (2 ** 32))
            stride = (stride * (resolution + 1)) % (2 ** 32)
        hashed = stride > size
        if hashed:
            c1 = int(np.int32(np.uint32(2654435761)))
            c2 = int(np.int32(np.uint32(805459861)))
            mode_add = False
        else:
            assert all(use_stride)
            c1 = int(np.int32(np.uint32(coeffs[1])))
            c2 = int(np.int32(np.uint32(coeffs[2])))
            mode_add = True
        params.append(dict(scale=float(scale), mask=size - 1,
                           off=offsets[lvl], add=mode_add, c1=c1, c2=c2))
    return params


_LEVELS = _level_tables()


def _make_relayout_kernel(nwords):
    f32 = jnp.float32
    i32 = jnp.int32
    wpt = nwords // NW
    CH = 8192
    nfull = wpt // CH
    tail = wpt - nfull * CH
    assert nfull >= 2 and nfull % 2 == 0 and tail % 256 == 0

    def body(fh, outh, in0, in1, ou0, ou1, si0, si1, so0, so1):
        cid = lax.axis_index("c")
        sid = lax.axis_index("s")
        wid = sid * NC + cid
        base = wid * wpt
        iota = lax.iota(i32, LANES)
        inb = (in0, in1)
        oub = (ou0, ou1)
        sin = (si0, si1)
        sou = (so0, so1)

        def transform(src, dst, ngroups):
            def tb(g, c):
                w0 = g * LANES
                wv = w0 + iota
                bv = wv & i32(-256)
                cv = wv & i32(1)
                rv = (wv & i32(255)) >> 1
                srcv = bv + (cv << i32(7)) + rv
                dst[pl.ds(w0, LANES)] = plsc.load_gather(src, [srcv])
                return c
            lax.fori_loop(0, ngroups, tb, 0)

        def drain_in(p):
            pltpu.make_async_copy(fh.at[pl.ds(base, CH)], inb[p],
                                  sin[p]).wait()

        def drain_out(p):
            pltpu.make_async_copy(oub[p], outh.at[pl.ds(base, CH)],
                                  sou[p]).wait()

        def fire_in(p, ck):
            pltpu.async_copy(fh.at[pl.ds(base + ck * CH, CH)], inb[p], sin[p])

        def fire_out(p, ck):
            pltpu.async_copy(oub[p], outh.at[pl.ds(base + ck * CH, CH)],
                             sou[p])

        fire_in(0, 0)
        drain_in(0)
        fire_in(1, 1)
        transform(in0, ou0, CH // LANES)
        fire_out(0, 0)
        drain_in(1)
        fire_in(0, 2)
        transform(in1, ou1, CH // LANES)
        fire_out(1, 1)

        def pair(k, carry):
            drain_in(0)
            fire_in(1, 2 * k + 1)
            drain_out(0)
            transform(in0, ou0, CH // LANES)
            fire_out(0, 2 * k)
            drain_in(1)
            nxt = lax.min(2 * k + 2, i32(nfull - 1))
            pltpu.async_copy(fh.at[pl.ds(base + nxt * CH, CH)], in0, si0)
            drain_out(1)
            transform(in1, ou1, CH // LANES)
            fire_out(1, 2 * k + 1)
            return carry

        lax.fori_loop(1, nfull // 2, pair, 0)
        drain_in(0)
        drain_out(0)
        drain_out(1)
        if tail:
            pltpu.sync_copy(fh.at[pl.ds(base + nfull * CH, tail)],
                            in0.at[pl.ds(0, tail)])
            transform(in0, ou0, tail // LANES)
            pltpu.sync_copy(ou0.at[pl.ds(0, tail)],
                            outh.at[pl.ds(base + nfull * CH, tail)])

    mesh = plsc.VectorSubcoreMesh(core_axis_name="c", subcore_axis_name="s")
    return pl.kernel(
        body,
        out_type=jax.ShapeDtypeStruct((nwords,), f32),
        mesh=mesh,
        compiler_params=pltpu.CompilerParams(
            needs_layout_passes=False,
            use_tc_tiling_on_sc=False,
        ),
        scratch_types=[
            pltpu.VMEM((CH,), f32),
            pltpu.VMEM((CH,), f32),
            pltpu.VMEM((CH,), f32),
            pltpu.VMEM((CH,), f32),
            pltpu.SemaphoreType.DMA,
            pltpu.SemaphoreType.DMA,
            pltpu.SemaphoreType.DMA,
            pltpu.SemaphoreType.DMA,
        ],
    )


def _make_grid_kernel(batch):
    chunk = batch // NW
    n = 256
    assert chunk % n == 0
    nsub = chunk // n
    groups = n // LANES
    m = 8 * n // 128

    f32 = jnp.float32
    i32 = jnp.int32

    def body(xh, yh, zh, eh, oh,
             xb, yb, zb, fb, idx0, idx1, lo0, lo1, rows0, rows1, accb, l0b,
             gsem0, gsem1, gsem2, gsem3, osem0):
        cid = lax.axis_index("c")
        sid = lax.axis_index("s")
        wid = sid * NC + cid
        base_w = wid * chunk
        iota = lax.iota(i32, LANES)
        idxb = (idx0, idx1)
        lob = (lo0, lo1)
        rowsb = (rows0, rows1)
        gsem = (gsem0, gsem1)
        gsemb = (gsem2, gsem3)

        pltpu.sync_copy(xh.at[pl.ds(base_w, chunk)], xb)
        pltpu.sync_copy(yh.at[pl.ds(base_w, chunk)], yb)
        pltpu.sync_copy(zh.at[pl.ds(base_w, chunk)], zb)
        pltpu.sync_copy(eh.at[pl.ds(0, 1024)], l0b)
        pltpu.async_copy(accb, oh.at[pl.ds(0, 2 * NUM_LEVELS),
                                     pl.ds(base_w, n)], osem0)

        def tbody(g, c):
            o = g * LANES
            for ref in (xb, yb, zb):
                v = ref[pl.ds(o, LANES)]
                ref[pl.ds(o, LANES)] = (v + f32(1.0)) * f32(0.5)
            return c
        lax.fori_loop(0, chunk // LANES, tbody, 0)

        def subchunk(s, carry):
            base = base_w + s * n
            sbase = s * n

            def compute_idx(lvl, p):
                prm = _LEVELS[lvl]
                scale = f32(prm["scale"])
                half = f32(0.5)
                c1 = i32(prm["c1"])
                c2 = i32(prm["c2"])
                mask = i32(prm["mask"])
                off = i32(prm["off"])
                iref = idxb[p]
                lref = lob[p]

                def cbody(g, c):
                    o = g * LANES
                    px = xb[pl.ds(sbase + o, LANES)] * scale + half
                    py = yb[pl.ds(sbase + o, LANES)] * scale + half
                    pz = zb[pl.ds(sbase + o, LANES)] * scale + half
                    pix = px.astype(i32)
                    piy = py.astype(i32)
                    piz = pz.astype(i32)
                    fb[p, 0, pl.ds(o, LANES)] = px - pix.astype(f32)
                    fb[p, 1, pl.ds(o, LANES)] = py - piy.astype(f32)
                    fb[p, 2, pl.ds(o, LANES)] = pz - piz.astype(f32)
                    ax = (pix, pix + i32(1))
                    by = (piy * c1, piy * c1 + c1)
                    cz = (piz * c2, piz * c2 + c2)
                    for c8 in range(8):
                        a = ax[c8 & 1]
                        b = by[(c8 >> 1) & 1]
                        cc = cz[(c8 >> 2) & 1]
                        if prm["add"]:
                            h = a + b + cc
                        else:
                            h = a ^ b ^ cc
                        glob = (h & mask) + off
                        iref[pl.ds(c8 * n + o, LANES)] = glob >> 2
                        lref[pl.ds(c8 * n + o, LANES)] = (glob & i32(3)) * 2
                    return c
                lax.fori_loop(0, groups, cbody, 0)

            def accumulate(lvl, p):
                rref = rowsb[p]
                lref = lob[p]
                iref = idxb[p]

                def abody(g, c):
                    o = g * LANES
                    fx = fb[p, 0, pl.ds(o, LANES)]
                    fy = fb[p, 1, pl.ds(o, LANES)]
                    fz = fb[p, 2, pl.ds(o, LANES)]
                    gx = f32(1.0) - fx
                    gy = f32(1.0) - fy
                    gz = f32(1.0) - fz
                    wxy = (gx * gy, fx * gy, gx * fy, fx * fy)
                    jv = o + iota
                    a0 = None
                    a1 = None
                    for c8 in range(8):
                        w = wxy[c8 & 3] * (gz if c8 < 4 else fz)
                        lo = lref[pl.ds(c8 * n + o, LANES)]
                        if lvl == 0:
                            pv = iref[pl.ds(c8 * n + o, LANES)]
                            v0 = plsc.load_gather(l0b, [pv, lo])
                            v1 = plsc.load_gather(l0b, [pv, lo + i32(1)])
                        else:
                            v0 = plsc.load_gather(rref, [jv + c8 * n, lo])
                            v1 = plsc.load_gather(rref,
                                                  [jv + c8 * n, lo + i32(1)])
                        t0 = w * v0
                        t1 = w * v1
                        a0 = t0 if a0 is None else a0 + t0
                        a1 = t1 if a1 is None else a1 + t1
                    accb[2 * lvl, pl.ds(o, LANES)] = a0
                    accb[2 * lvl + 1, pl.ds(o, LANES)] = a1
                    return c
                lax.fori_loop(0, groups, abody, 0)

            ghandles = [None, None]
            for lvl in range(NUM_LEVELS):
                p = lvl & 1
                compute_idx(lvl, p)
                if lvl > 0:
                    ha = pltpu.async_copy(
                        eh.at[idxb[p].at[pl.ds(0, 4 * n)]],
                        rowsb[p].at[pl.ds(0, 4 * n)], gsem[p])
                    hb = pltpu.async_copy(
                        eh.at[idxb[p].at[pl.ds(4 * n, 4 * n)]],
                        rowsb[p].at[pl.ds(4 * n, 4 * n)], gsemb[p])
                    ghandles[p] = (ha, hb)
                if lvl == 1:
                    pltpu.make_async_copy(
                        accb, oh.at[pl.ds(0, 2 * NUM_LEVELS), pl.ds(base, n)],
                        osem0).wait()
                    accumulate(0, 0)
                elif lvl > 1:
                    q = (lvl - 1) & 1
                    ghandles[q][0].wait()
                    ghandles[q][1].wait()
                    accumulate(lvl - 1, q)
            ghandles[1][0].wait()
            ghandles[1][1].wait()
            accumulate(NUM_LEVELS - 1, 1)
            pltpu.async_copy(
                accb, oh.at[pl.ds(0, 2 * NUM_LEVELS), pl.ds(base, n)],
                osem0)
            return carry

        lax.fori_loop(0, nsub, subchunk, 0)
        pltpu.make_async_copy(
            accb,
            oh.at[pl.ds(0, 2 * NUM_LEVELS),
                  pl.ds(base_w + (nsub - 1) * n, n)],
            osem0).wait()

    mesh = plsc.VectorSubcoreMesh(core_axis_name="c", subcore_axis_name="s")
    return pl.kernel(
        body,
        out_type=jax.ShapeDtypeStruct((NUM_LEVELS * LEVEL_DIM, batch), f32),
        mesh=mesh,
        compiler_params=pltpu.CompilerParams(
            needs_layout_passes=False,
            use_tc_tiling_on_sc=False,
        ),
        scratch_types=[
            pltpu.VMEM((chunk,), f32),
            pltpu.VMEM((chunk,), f32),
            pltpu.VMEM((chunk,), f32),
            pltpu.VMEM((2, 3, n), f32),
            pltpu.VMEM((8 * n,), i32),
            pltpu.VMEM((8 * n,), i32),
            pltpu.VMEM((8 * n,), i32),
            pltpu.VMEM((8 * n,), i32),
            pltpu.VMEM((8 * n, 8), f32),
            pltpu.VMEM((8 * n, 8), f32),
            pltpu.VMEM((2 * NUM_LEVELS, n), f32),
            pltpu.VMEM((1024, 8), f32),
            pltpu.SemaphoreType.DMA,
            pltpu.SemaphoreType.DMA,
            pltpu.SemaphoreType.DMA,
            pltpu.SemaphoreType.DMA,
            pltpu.SemaphoreType.DMA,
        ],
    )


@jax.jit
def kernel(inputs, embeddings):
    batch = inputs.shape[0]
    xt = inputs.T
    nrows = embeddings.shape[0]
    emb3 = jnp.transpose(embeddings.reshape(nrows // 128, 128, 2), (0, 2, 1))
    flat = emb3.reshape(nrows * 2)
    rowmajor = _make_relayout_kernel(nrows * 2)(flat)
    packed = rowmajor.reshape(nrows * 2 // 8, 8)
    grid = _make_grid_kernel(batch)
    out = grid(xt[0], xt[1], xt[2], packed)
    return out.T
